# baseline (speedup 1.0000x reference)
import functools
import math

import jax
import jax.numpy as jnp
from jax import lax
from jax.experimental import pallas as pl
from jax.experimental.pallas import tpu as pltpu
from jax.experimental.pallas import tpu_sc as plsc

N = 131072
NC = 2
NS = 16
L = 16
NW = NC * NS
CHUNK = N // NW
NVEC = CHUNK // L

_LOG_HALF = math.log(0.5)
_LOG_2PI = math.log(2.0 * math.pi)
_OBS_STD = 0.1
_K_OBS = 0.5 / (_OBS_STD * _OBS_STD)
_K_1 = 0.5
_K_20 = 0.5
_K_21 = 0.5 / (1.5 * 1.5)
_C_1 = _LOG_HALF - math.log(1.0) - math.log(_OBS_STD) - _LOG_2PI
_C_20 = 2.0 * _LOG_HALF - math.log(1.0) - math.log(_OBS_STD) - _LOG_2PI
_C_21 = 2.0 * _LOG_HALF - math.log(1.5) - math.log(_OBS_STD) - _LOG_2PI

_MEAN_20 = -2.0
_MEAN_21 = 2.0

_mesh = plsc.VectorSubcoreMesh(
    core_axis_name="c", subcore_axis_name="s", num_cores=NC, num_subcores=NS
)

_f32 = jnp.float32


@functools.partial(
    pl.kernel,
    out_type=(
        jax.ShapeDtypeStruct((N,), _f32),
        jax.ShapeDtypeStruct((N,), _f32),
        jax.ShapeDtypeStruct((N,), _f32),
    ),
    mesh=_mesh,
    scratch_types=[
        pltpu.VMEM((CHUNK,), _f32),
        pltpu.VMEM((CHUNK,), _f32),
        pltpu.VMEM((CHUNK,), _f32),
        pltpu.VMEM((CHUNK,), _f32),
        pltpu.VMEM((CHUNK,), _f32),
        pltpu.VMEM((CHUNK,), _f32),
        pltpu.VMEM((CHUNK,), _f32),
        pltpu.VMEM((CHUNK,), _f32),
        pltpu.VMEM((CHUNK,), _f32),
        pltpu.VMEM((L,), _f32),
        pltpu.SemaphoreType.DMA,
    ],
)
def _sc_logpdf(x1_h, o1_h, x20_h, o20_h, x21_h, o21_h, mean_h,
               y1_h, y20_h, y21_h,
               x1_v, o1_v, x20_v, o20_v, x21_v, o21_v,
               y1_v, y20_v, y21_v, mean_v, sem):
    wid = lax.axis_index("s") * NC + lax.axis_index("c")
    base = wid * CHUNK
    sl = pl.ds(base, CHUNK)

    copies = [
        pltpu.async_copy(mean_h, mean_v, sem),
        pltpu.async_copy(x1_h.at[sl], x1_v, sem),
        pltpu.async_copy(o1_h.at[sl], o1_v, sem),
        pltpu.async_copy(x20_h.at[sl], x20_v, sem),
        pltpu.async_copy(o20_h.at[sl], o20_v, sem),
        pltpu.async_copy(x21_h.at[sl], x21_v, sem),
        pltpu.async_copy(o21_h.at[sl], o21_v, sem),
    ]
    for c in copies:
        c.wait()

    m = mean_v[...]

    def step(i, _):
        s = pl.ds(i * L, L)
        x = x1_v[s]
        o = o1_v[s]
        d = x - m
        e = o - x
        y1_v[s] = _C_1 - _K_1 * (d * d) - _K_OBS * (e * e)
        x = x20_v[s]
        o = o20_v[s]
        d = x - _MEAN_20
        e = o - x
        y20_v[s] = _C_20 - _K_20 * (d * d) - _K_OBS * (e * e)
        x = x21_v[s]
        o = o21_v[s]
        d = x - _MEAN_21
        e = o - x
        y21_v[s] = _C_21 - _K_21 * (d * d) - _K_OBS * (e * e)
        return 0

    lax.fori_loop(0, NVEC, step, 0)

    outs = [
        pltpu.async_copy(y1_v, y1_h.at[sl], sem),
        pltpu.async_copy(y20_v, y20_h.at[sl], sem),
        pltpu.async_copy(y21_v, y21_h.at[sl], sem),
    ]
    for c in outs:
        c.wait()


def kernel(k_1, x_1, obs_1, k_20, z_0, x_20, obs_20, k_21, z_1, x_21, obs_21,
           mean_1):
    del k_1, k_20, z_0, k_21, z_1
    mean_lanes = jnp.broadcast_to(mean_1.astype(_f32), (L,))
    out_1, out_20, out_21 = _sc_logpdf(
        x_1, obs_1, x_20, obs_20, x_21, obs_21, mean_lanes
    )
    return (out_1, out_20, out_21)
